# Initial kernel scaffold; baseline (speedup 1.0000x reference)
import jax
import jax.numpy as jnp
from jax.experimental import pallas as pl


def kernel(x, edge_index, batch, W0_0, W0_1, W0_2, b0, W1_0, W1_1, W1_2, b1):
    raise NotImplementedError("write your pallas kernel here")


# speedup vs baseline: 2.3299x; 2.3299x over previous
import functools

import jax
import jax.numpy as jnp
from jax import lax
from jax.experimental import pallas as pl
from jax.experimental.pallas import tpu as pltpu
from jax.experimental.pallas import tpu_sc as plsc

_N = 10000
_E = 320000
_D = 128
_CHUNK = 128
_NW = 32
_E_PAD = 327680
_CHUNKS = _E_PAD // _CHUNK
_CPW = _CHUNKS // _NW
_ROWS_PER_TILE = 640
_N_PAD = 10240
_TRASH = _N


def _hop(src, rows_c, cols_c, zeros):
    mesh = plsc.VectorSubcoreMesh(core_axis_name="c", subcore_axis_name="s")

    @functools.partial(
        pl.kernel,
        out_type=jax.ShapeDtypeStruct((2, _N_PAD, _D), jnp.float32),
        mesh=mesh,
        scratch_types=[
            pltpu.VMEM((_CHUNK,), jnp.int32),
            pltpu.VMEM((_CHUNK,), jnp.int32),
            pltpu.VMEM((_CHUNK, _D), jnp.float32),
            pltpu.VMEM_SHARED((_N_PAD, _D), jnp.float32),
            pltpu.SemaphoreType.DMA,
        ],
    )
    def hop_kernel(src_hbm, rows_hbm, cols_hbm, zeros_hbm, out_hbm,
                   col_v, row_v, gath_v, acc_sh, sem):
        c = lax.axis_index("c")
        s = lax.axis_index("s")
        wid = s * 2 + c

        pltpu.sync_copy(zeros_hbm, acc_sh.at[pl.ds(s * _ROWS_PER_TILE,
                                                   _ROWS_PER_TILE)])
        plsc.subcore_barrier()

        def body(i, carry):
            ch = wid * _CPW + i
            pltpu.sync_copy(cols_hbm.at[ch], col_v)
            pltpu.async_copy(src_hbm.at[col_v], gath_v, sem).wait()
            pltpu.sync_copy(rows_hbm.at[ch], row_v)
            pltpu.sync_copy(gath_v, acc_sh.at[row_v], add=True)
            return carry

        lax.fori_loop(0, _CPW, body, 0)
        plsc.subcore_barrier()

        pltpu.sync_copy(acc_sh.at[pl.ds(s * _ROWS_PER_TILE, _ROWS_PER_TILE)],
                        out_hbm.at[c, pl.ds(s * _ROWS_PER_TILE,
                                            _ROWS_PER_TILE)])

    return hop_kernel(src, rows_c, cols_c, zeros)


def _combine_body(p0_ref, p1_ref, o_ref):
    o_ref[...] = p0_ref[0] + p1_ref[0]


def _combine(p):
    blk = 1024
    return pl.pallas_call(
        _combine_body,
        grid=(_N_PAD // blk,),
        in_specs=[
            pl.BlockSpec((1, blk, _D), lambda i: (0, i, 0)),
            pl.BlockSpec((1, blk, _D), lambda i: (1, i, 0)),
        ],
        out_specs=pl.BlockSpec((blk, _D), lambda i: (i, 0)),
        out_shape=jax.ShapeDtypeStruct((_N_PAD, _D), jnp.float32),
    )(p, p)


def _final_body(x_ref, x1_ref, q0_ref, q1_ref, w0_ref, w1_ref, w2_ref, b_ref,
                o_ref):
    x2 = q0_ref[0] + q1_ref[0]
    acc = jnp.dot(x_ref[...], w0_ref[...], preferred_element_type=jnp.float32)
    acc = acc + jnp.dot(x1_ref[...], w1_ref[...],
                        preferred_element_type=jnp.float32)
    acc = acc + jnp.dot(x2, w2_ref[...], preferred_element_type=jnp.float32)
    o_ref[...] = acc + b_ref[...]


def _final(x, x1, q, w0, w1, w2, b):
    blk = 1000
    return pl.pallas_call(
        _final_body,
        grid=(_N // blk,),
        in_specs=[
            pl.BlockSpec((blk, _D), lambda i: (i, 0)),
            pl.BlockSpec((blk, _D), lambda i: (i, 0)),
            pl.BlockSpec((1, blk, _D), lambda i: (0, i, 0)),
            pl.BlockSpec((1, blk, _D), lambda i: (1, i, 0)),
            pl.BlockSpec((_D, _D), lambda i: (0, 0)),
            pl.BlockSpec((_D, _D), lambda i: (0, 0)),
            pl.BlockSpec((_D, _D), lambda i: (0, 0)),
            pl.BlockSpec((1, _D), lambda i: (0, 0)),
        ],
        out_specs=pl.BlockSpec((blk, _D), lambda i: (i, 0)),
        out_shape=jax.ShapeDtypeStruct((_N, _D), jnp.float32),
    )(x, x1, q, q, w0, w1, w2, b)


def kernel(x, edge_index, batch, W0_0, W0_1, W0_2, b0, W1_0, W1_1, W1_2, b1):
    rows = edge_index[0]
    cols = edge_index[1]
    pad = _E_PAD - _E
    rows_c = jnp.concatenate(
        [rows, jnp.full((pad,), _TRASH, jnp.int32)]).reshape(_CHUNKS, _CHUNK)
    cols_c = jnp.concatenate(
        [cols, jnp.zeros((pad,), jnp.int32)]).reshape(_CHUNKS, _CHUNK)
    zeros = jnp.zeros((_ROWS_PER_TILE, _D), jnp.float32)

    p = _hop(x, rows_c, cols_c, zeros)
    x1 = _combine(p)
    q = _hop(x1, rows_c, cols_c, zeros)
    return _final(x, x1, q, W1_0, W1_1, W1_2, b1.reshape(1, _D))


# speedup vs baseline: 3.1184x; 1.3384x over previous
import functools

import jax
import jax.numpy as jnp
from jax import lax
from jax.experimental import pallas as pl
from jax.experimental.pallas import tpu as pltpu
from jax.experimental.pallas import tpu_sc as plsc

_N = 10000
_E = 320000
_D = 128
_CHUNK = 128
_NW = 32
_E_PAD = 327680
_CHUNKS = _E_PAD // _CHUNK
_CPW = _CHUNKS // _NW
_CPH = _CPW // 2
_ROWS_PER_TILE = 632
_N_PAD = 10112
_TRASH = _N


def _hop(src, rows_c, cols_c, zeros):
    mesh = plsc.VectorSubcoreMesh(core_axis_name="c", subcore_axis_name="s")

    @functools.partial(
        pl.kernel,
        out_type=jax.ShapeDtypeStruct((2, _N_PAD, _D), jnp.float32),
        mesh=mesh,
        scratch_types=[
            pltpu.VMEM((_CPH, _CHUNK), jnp.int32),
            pltpu.VMEM((_CPH, _CHUNK), jnp.int32),
            pltpu.VMEM((_CHUNK, _D), jnp.float32),
            pltpu.VMEM((_CHUNK, _D), jnp.float32),
            pltpu.VMEM_SHARED((_N_PAD, _D), jnp.float32),
            pltpu.SemaphoreType.DMA,
            pltpu.SemaphoreType.DMA,
        ],
    )
    def hop_kernel(src_hbm, rows_hbm, cols_hbm, zeros_hbm, out_hbm,
                   col_v, row_v, gath0_v, gath1_v, acc_sh, sem0, sem1):
        c = lax.axis_index("c")
        s = lax.axis_index("s")
        wid = s * 2 + c

        pltpu.sync_copy(zeros_hbm, acc_sh.at[pl.ds(s * _ROWS_PER_TILE,
                                                   _ROWS_PER_TILE)])
        plsc.subcore_barrier()

        bufs = (gath0_v, gath1_v)
        sems = (sem0, sem1)

        for half in range(2):
            base = wid * _CPW + half * _CPH
            pltpu.sync_copy(cols_hbm.at[pl.ds(base, _CPH)], col_v)
            pltpu.sync_copy(rows_hbm.at[pl.ds(base, _CPH)], row_v)

            pltpu.async_copy(src_hbm.at[col_v.at[0]], bufs[0], sems[0])

            def body2(k2, carry):
                k = 2 * k2

                @pl.when(k + 1 < _CPH)
                def _():
                    pltpu.async_copy(src_hbm.at[col_v.at[k + 1]],
                                     bufs[1], sems[1])
                pltpu.make_async_copy(src_hbm.at[col_v.at[k]],
                                      bufs[0], sems[0]).wait()
                pltpu.sync_copy(bufs[0], acc_sh.at[row_v.at[k]], add=True)

                @pl.when(k + 2 < _CPH)
                def _():
                    pltpu.async_copy(src_hbm.at[col_v.at[k + 2]],
                                     bufs[0], sems[0])
                pltpu.make_async_copy(src_hbm.at[col_v.at[k + 1]],
                                      bufs[1], sems[1]).wait()
                pltpu.sync_copy(bufs[1], acc_sh.at[row_v.at[k + 1]], add=True)
                return carry

            lax.fori_loop(0, _CPH // 2, body2, 0)
        plsc.subcore_barrier()

        pltpu.sync_copy(acc_sh.at[pl.ds(s * _ROWS_PER_TILE, _ROWS_PER_TILE)],
                        out_hbm.at[c, pl.ds(s * _ROWS_PER_TILE,
                                            _ROWS_PER_TILE)])

    return hop_kernel(src, rows_c, cols_c, zeros)


def _combine_body(p0_ref, p1_ref, o_ref):
    o_ref[...] = p0_ref[0] + p1_ref[0]


def _combine(p):
    blk = 1264
    return pl.pallas_call(
        _combine_body,
        grid=(_N_PAD // blk,),
        in_specs=[
            pl.BlockSpec((1, blk, _D), lambda i: (0, i, 0)),
            pl.BlockSpec((1, blk, _D), lambda i: (1, i, 0)),
        ],
        out_specs=pl.BlockSpec((blk, _D), lambda i: (i, 0)),
        out_shape=jax.ShapeDtypeStruct((_N_PAD, _D), jnp.float32),
    )(p, p)


def _final_body(x_ref, x1_ref, q0_ref, q1_ref, w0_ref, w1_ref, w2_ref, b_ref,
                o_ref):
    x2 = q0_ref[0] + q1_ref[0]
    acc = jnp.dot(x_ref[...], w0_ref[...], preferred_element_type=jnp.float32)
    acc = acc + jnp.dot(x1_ref[...], w1_ref[...],
                        preferred_element_type=jnp.float32)
    acc = acc + jnp.dot(x2, w2_ref[...], preferred_element_type=jnp.float32)
    o_ref[...] = acc + b_ref[...]


def _final(x, x1, q, w0, w1, w2, b):
    blk = 1000
    return pl.pallas_call(
        _final_body,
        grid=(_N // blk,),
        in_specs=[
            pl.BlockSpec((blk, _D), lambda i: (i, 0)),
            pl.BlockSpec((blk, _D), lambda i: (i, 0)),
            pl.BlockSpec((1, blk, _D), lambda i: (0, i, 0)),
            pl.BlockSpec((1, blk, _D), lambda i: (1, i, 0)),
            pl.BlockSpec((_D, _D), lambda i: (0, 0)),
            pl.BlockSpec((_D, _D), lambda i: (0, 0)),
            pl.BlockSpec((_D, _D), lambda i: (0, 0)),
            pl.BlockSpec((1, _D), lambda i: (0, 0)),
        ],
        out_specs=pl.BlockSpec((blk, _D), lambda i: (i, 0)),
        out_shape=jax.ShapeDtypeStruct((_N, _D), jnp.float32),
    )(x, x1, q, q, w0, w1, w2, b)


def kernel(x, edge_index, batch, W0_0, W0_1, W0_2, b0, W1_0, W1_1, W1_2, b1):
    rows = edge_index[0]
    cols = edge_index[1]
    pad = _E_PAD - _E
    rows_c = jnp.concatenate(
        [rows, jnp.full((pad,), _TRASH, jnp.int32)]).reshape(_CHUNKS, _CHUNK)
    cols_c = jnp.concatenate(
        [cols, jnp.zeros((pad,), jnp.int32)]).reshape(_CHUNKS, _CHUNK)
    zeros = jnp.zeros((_ROWS_PER_TILE, _D), jnp.float32)

    p = _hop(x, rows_c, cols_c, zeros)
    x1 = _combine(p)
    q = _hop(x1, rows_c, cols_c, zeros)
    return _final(x, x1, q, W1_0, W1_1, W1_2, b1.reshape(1, _D))


# speedup vs baseline: 3.3621x; 1.0781x over previous
import functools

import jax
import jax.numpy as jnp
from jax import lax
from jax.experimental import pallas as pl
from jax.experimental.pallas import tpu as pltpu
from jax.experimental.pallas import tpu_sc as plsc

_N = 10000
_E = 320000
_D = 128
_CHUNK = 128
_NW = 32
_E_PAD = 327680
_CHUNKS = _E_PAD // _CHUNK
_STAGE = 32
_CPW0 = 128
_CPW1 = 32
_ROWS_PER_TILE = 632
_N_PAD = 10112
_TRASH = _N


def _hop(src, rows_c, cols_c, zeros):
    mesh = plsc.VectorSubcoreMesh(core_axis_name="c", subcore_axis_name="s")

    @functools.partial(
        pl.kernel,
        out_type=jax.ShapeDtypeStruct((2, _N_PAD, _D), jnp.float32),
        mesh=mesh,
        scratch_types=[
            pltpu.VMEM((_STAGE, _CHUNK), jnp.int32),
            pltpu.VMEM((_STAGE, _CHUNK), jnp.int32),
            pltpu.VMEM((_CHUNK, _D), jnp.float32),
            pltpu.VMEM((_CHUNK, _D), jnp.float32),
            pltpu.VMEM_SHARED((_N_PAD, _D), jnp.float32),
            pltpu.SemaphoreType.DMA,
            pltpu.SemaphoreType.DMA,
        ],
    )
    def hop_kernel(src_hbm, rows_hbm, cols_hbm, zeros_hbm, out_hbm,
                   col_v, row_v, gath0_v, gath1_v, acc_sh, sem0, sem1):
        c = lax.axis_index("c")
        s = lax.axis_index("s")

        pltpu.sync_copy(zeros_hbm, acc_sh.at[pl.ds(s * _ROWS_PER_TILE,
                                                   _ROWS_PER_TILE)])
        plsc.subcore_barrier()

        bufs = (gath0_v, gath1_v)
        sems = (sem0, sem1)

        def run_stage(base):
            pltpu.sync_copy(cols_hbm.at[pl.ds(base, _STAGE)], col_v)
            pltpu.sync_copy(rows_hbm.at[pl.ds(base, _STAGE)], row_v)

            pltpu.async_copy(src_hbm.at[col_v.at[0]], bufs[0], sems[0])

            def body2(k2, carry):
                k = 2 * k2

                @pl.when(k + 1 < _STAGE)
                def _():
                    pltpu.async_copy(src_hbm.at[col_v.at[k + 1]],
                                     bufs[1], sems[1])
                pltpu.make_async_copy(src_hbm.at[col_v.at[k]],
                                      bufs[0], sems[0]).wait()
                pltpu.sync_copy(bufs[0], acc_sh.at[row_v.at[k]], add=True)

                @pl.when(k + 2 < _STAGE)
                def _():
                    pltpu.async_copy(src_hbm.at[col_v.at[k + 2]],
                                     bufs[0], sems[0])
                pltpu.make_async_copy(src_hbm.at[col_v.at[k + 1]],
                                      bufs[1], sems[1]).wait()
                pltpu.sync_copy(bufs[1], acc_sh.at[row_v.at[k + 1]], add=True)
                return carry

            lax.fori_loop(0, _STAGE // 2, body2, 0)

        @pl.when(c == 0)
        def _():
            for st in range(_CPW0 // _STAGE):
                run_stage(s * _CPW0 + st * _STAGE)

        @pl.when(c == 1)
        def _():
            for st in range(_CPW1 // _STAGE):
                run_stage(16 * _CPW0 + s * _CPW1 + st * _STAGE)

        plsc.subcore_barrier()

        pltpu.sync_copy(acc_sh.at[pl.ds(s * _ROWS_PER_TILE, _ROWS_PER_TILE)],
                        out_hbm.at[c, pl.ds(s * _ROWS_PER_TILE,
                                            _ROWS_PER_TILE)])

    return hop_kernel(src, rows_c, cols_c, zeros)


def _combine_body(p0_ref, p1_ref, o_ref):
    o_ref[...] = p0_ref[0] + p1_ref[0]


def _combine(p):
    blk = 1264
    return pl.pallas_call(
        _combine_body,
        grid=(_N_PAD // blk,),
        in_specs=[
            pl.BlockSpec((1, blk, _D), lambda i: (0, i, 0)),
            pl.BlockSpec((1, blk, _D), lambda i: (1, i, 0)),
        ],
        out_specs=pl.BlockSpec((blk, _D), lambda i: (i, 0)),
        out_shape=jax.ShapeDtypeStruct((_N_PAD, _D), jnp.float32),
    )(p, p)


def _final_body(x_ref, x1_ref, q0_ref, q1_ref, w0_ref, w1_ref, w2_ref, b_ref,
                o_ref):
    x2 = q0_ref[0] + q1_ref[0]
    acc = jnp.dot(x_ref[...], w0_ref[...], preferred_element_type=jnp.float32)
    acc = acc + jnp.dot(x1_ref[...], w1_ref[...],
                        preferred_element_type=jnp.float32)
    acc = acc + jnp.dot(x2, w2_ref[...], preferred_element_type=jnp.float32)
    o_ref[...] = acc + b_ref[...]


def _final(x, x1, q, w0, w1, w2, b):
    blk = 1000
    return pl.pallas_call(
        _final_body,
        grid=(_N // blk,),
        in_specs=[
            pl.BlockSpec((blk, _D), lambda i: (i, 0)),
            pl.BlockSpec((blk, _D), lambda i: (i, 0)),
            pl.BlockSpec((1, blk, _D), lambda i: (0, i, 0)),
            pl.BlockSpec((1, blk, _D), lambda i: (1, i, 0)),
            pl.BlockSpec((_D, _D), lambda i: (0, 0)),
            pl.BlockSpec((_D, _D), lambda i: (0, 0)),
            pl.BlockSpec((_D, _D), lambda i: (0, 0)),
            pl.BlockSpec((1, _D), lambda i: (0, 0)),
        ],
        out_specs=pl.BlockSpec((blk, _D), lambda i: (i, 0)),
        out_shape=jax.ShapeDtypeStruct((_N, _D), jnp.float32),
    )(x, x1, q, q, w0, w1, w2, b)


def kernel(x, edge_index, batch, W0_0, W0_1, W0_2, b0, W1_0, W1_1, W1_2, b1):
    rows = edge_index[0]
    cols = edge_index[1]
    pad = _E_PAD - _E
    rows_c = jnp.concatenate(
        [rows, jnp.full((pad,), _TRASH, jnp.int32)]).reshape(_CHUNKS, _CHUNK)
    cols_c = jnp.concatenate(
        [cols, jnp.zeros((pad,), jnp.int32)]).reshape(_CHUNKS, _CHUNK)
    zeros = jnp.zeros((_ROWS_PER_TILE, _D), jnp.float32)

    p = _hop(x, rows_c, cols_c, zeros)
    x1 = _combine(p)
    q = _hop(x1, rows_c, cols_c, zeros)
    return _final(x, x1, q, W1_0, W1_1, W1_2, b1.reshape(1, _D))
